# Initial kernel scaffold; baseline (speedup 1.0000x reference)
import jax
import jax.numpy as jnp
from jax.experimental import pallas as pl


def kernel(x, glove_table, W, b):
    raise NotImplementedError("write your pallas kernel here")


# speedup vs baseline: 1.8811x; 1.8811x over previous
import functools

import jax
import jax.numpy as jnp
from jax import lax
from jax.experimental import pallas as pl
from jax.experimental.pallas import tpu as pltpu
from jax.experimental.pallas import tpu_sc as plsc

GLOVE_DIM = 300
D_MODEL = 768
DIM_MAIN = 256
DIM_PAD = 384


def _make_sc_gather(num_rows: int):
    info = plsc.get_sparse_core_info()
    nc, ns = info.num_cores, info.num_subcores
    nw = nc * ns
    assert num_rows % (8 * nw) == 0
    b_per_w = num_rows // nw
    chunk = 80
    assert b_per_w % chunk == 0 and chunk % 8 == 0
    n_chunks = b_per_w // chunk

    mesh = plsc.VectorSubcoreMesh(core_axis_name="c", subcore_axis_name="s")

    @functools.partial(
        pl.kernel,
        mesh=mesh,
        out_type=jax.ShapeDtypeStruct((num_rows, DIM_PAD), jnp.float32),
        scratch_types=[
            pltpu.VMEM((2, chunk), jnp.int32),
            pltpu.VMEM((2, chunk, DIM_PAD), jnp.float32),
            pltpu.SemaphoreType.DMA,
            pltpu.SemaphoreType.DMA,
        ],
    )
    def gather(main_hbm, tail_hbm, idx_hbm, out_hbm, idx_v, rows_v, sem0, sem1):
        wid = lax.axis_index("s") * nc + lax.axis_index("c")
        base = wid * b_per_w
        sems = (sem0, sem1)

        def fire(g, buf):
            off = base + g * chunk
            pltpu.sync_copy(idx_hbm.at[pl.ds(off, chunk)], idx_v.at[buf])
            for t in range(DIM_MAIN // 128):
                pltpu.async_copy(
                    main_hbm.at[idx_v.at[buf], pl.ds(t * 128, 128)],
                    rows_v.at[buf, :, pl.ds(t * 128, 128)],
                    sems[buf],
                )
            pltpu.async_copy(
                tail_hbm.at[idx_v.at[buf]],
                rows_v.at[buf, :, pl.ds(DIM_MAIN, 128)],
                sems[buf],
            )

        def drain_write(g, buf):
            for _ in range(DIM_MAIN // 128 + 1):
                pltpu.make_async_copy(
                    tail_hbm.at[idx_v.at[buf]],
                    rows_v.at[buf, :, pl.ds(DIM_MAIN, 128)],
                    sems[buf],
                ).wait()
            pltpu.sync_copy(rows_v.at[buf], out_hbm.at[pl.ds(base + g * chunk, chunk)])

        fire(0, 0)

        def body(t, _):
            g = 2 * t

            @pl.when(g + 1 < n_chunks)
            def _():
                fire(g + 1, 1)

            drain_write(g, 0)

            @pl.when(g + 1 < n_chunks)
            def _():
                @pl.when(g + 2 < n_chunks)
                def _():
                    fire(g + 2, 0)

                drain_write(g + 1, 1)

            return 0

        lax.fori_loop(0, (n_chunks + 1) // 2, body, 0)

    return gather


def _mm_body(a_ref, w_ref, b_ref, o_ref):
    o_ref[...] = (
        jnp.dot(a_ref[...], w_ref[...], preferred_element_type=jnp.float32)
        + b_ref[...]
    )


def _matmul_tc(emb, wp, b):
    m = emb.shape[0]
    bm = 512
    return pl.pallas_call(
        _mm_body,
        grid=(m // bm,),
        in_specs=[
            pl.BlockSpec((bm, DIM_PAD), lambda i: (i, 0)),
            pl.BlockSpec((DIM_PAD, D_MODEL), lambda i: (0, 0)),
            pl.BlockSpec((1, D_MODEL), lambda i: (0, 0)),
        ],
        out_specs=pl.BlockSpec((bm, D_MODEL), lambda i: (i, 0)),
        out_shape=jax.ShapeDtypeStruct((m, D_MODEL), jnp.float32),
    )(emb, wp, b.reshape(1, D_MODEL))


def kernel(x, glove_table, W, b):
    batch, hist = x.shape
    idx = x.astype(jnp.int32).reshape(-1)
    tail = jnp.pad(glove_table[:, DIM_MAIN:], ((0, 0), (0, 128 - (GLOVE_DIM - DIM_MAIN))))
    wp = jnp.pad(W, ((0, DIM_PAD - GLOVE_DIM), (0, 0)))
    gather = _make_sc_gather(idx.shape[0])
    emb = gather(glove_table, tail, idx)
    out = _matmul_tc(emb, wp, b)
    return out.reshape(batch, hist, D_MODEL)
